# baseline (speedup 1.0000x reference)
import functools

import jax
import jax.numpy as jnp
from jax import lax
from jax.experimental import pallas as pl
from jax.experimental.pallas import tpu as pltpu, tpu_sc as plsc


def _make_sc_gather(V, D, B):
    info = plsc.get_sparse_core_info()
    L = info.num_lanes
    NW = info.num_cores * info.num_subcores
    assert B % (8 * NW) == 0 and D % L == 0
    b_per_w = B // NW
    NB = 8
    HC = b_per_w // 2
    mesh = plsc.VectorSubcoreMesh(core_axis_name="c", subcore_axis_name="s")

    @functools.partial(
        pl.kernel,
        mesh=mesh,
        compiler_params=pltpu.CompilerParams(needs_layout_passes=False),
        out_type=jax.ShapeDtypeStruct((B, 2 * D), jnp.float32),
        scratch_types=[
            pltpu.VMEM((b_per_w + L,), jnp.int32),
            pltpu.VMEM((b_per_w + L,), jnp.int32),
            pltpu.VMEM((b_per_w + L,), jnp.int32),
            pltpu.VMEM((b_per_w + L,), jnp.int32),
            pltpu.VMEM((HC,), jnp.int32),
            pltpu.VMEM((HC,), jnp.int32),
            pltpu.VMEM((NB, D, 128), jnp.float32),
            pltpu.VMEM((HC, 2 * D), jnp.float32),
            pltpu.SemaphoreType.DMA,
            pltpu.SemaphoreType.DMA,
        ],
    )
    def emb(slab_hbm, perm_hbm, tt_hbm, out_hbm,
            lab_v, labs_v, ord_v, wbase_v, pa_v, pb_v, buf_v, rows_v,
            sem, sem2):
        wid = lax.axis_index("s") * info.num_cores + lax.axis_index("c")
        base = wid * b_per_w
        pltpu.sync_copy(slab_hbm.at[pl.ds(base, b_per_w)],
                        lab_v.at[pl.ds(0, b_per_w)])
        pltpu.sync_copy(slab_hbm.at[pl.ds(base, b_per_w)],
                        labs_v.at[pl.ds(8, b_per_w)])
        pltpu.sync_copy(perm_hbm.at[pl.ds(base, HC)], pa_v)
        pltpu.sync_copy(perm_hbm.at[pl.ds(base + HC, HC)], pb_v)

        def scan_body(g, nwin):
            vec = lab_v[pl.ds(g * L, L)]
            win = lax.shift_right_logical(vec, 7)
            pvec = labs_v[pl.ds(g * L + 7, L)]
            pwin = lax.shift_right_logical(pvec, 7)
            first = jnp.logical_and(g == 0, lax.iota(jnp.int32, L) == 0)
            chg = jnp.logical_or(win != pwin, first)
            inc = jnp.where(chg, jnp.int32(1), jnp.int32(0))
            ord_v[pl.ds(g * L, L)] = nwin - 1 + plsc.cumsum(inc)
            plsc.store_compressed(
                wbase_v.at[pl.ds(nwin, L)], win * 128, mask=chg
            )
            cnt = plsc.all_reduce_population_count(chg)
            return nwin + cnt[0]

        n_win = lax.fori_loop(0, b_per_w // L, scan_body, jnp.int32(0))

        def fire(k):
            kc = jnp.minimum(k, n_win - 1)
            bse = pl.multiple_of(wbase_v[pl.ds(kc, L)][0], 128)
            pltpu.async_copy(
                tt_hbm.at[:, pl.ds(bse, 128)],
                buf_v.at[lax.rem(kc, NB)], sem
            )

        def drain():
            pltpu.make_async_copy(
                tt_hbm.at[:, pl.ds(0, 128)], buf_v.at[0], sem
            ).wait()

        for k in range(NB - 1):
            fire(jnp.int32(k))

        def label_body(i, cur):
            o = ord_v[pl.ds(i, L)][0]

            @pl.when(o != cur)
            def _():
                drain()
                fire(o + NB - 1)

            slot = lax.rem(o, NB)
            col = lab_v[pl.ds(i, L)][0]
            lo16 = jnp.full((L,), lax.rem(col, 128), jnp.int32)
            r = lax.rem(i, HC)
            for q in range(D // L):
                c16 = lax.iota(jnp.int32, L) + q * L
                rows_v[r, pl.ds(q * L, L)] = plsc.load_gather(
                    buf_v.at[slot], [c16, lo16]
                )
            return o

        cur = lax.fori_loop(0, HC, label_body, jnp.int32(-1))
        pltpu.async_copy(rows_v, out_hbm.at[pa_v], sem2).wait()
        cur = lax.fori_loop(HC, b_per_w, label_body, cur)
        pltpu.async_copy(rows_v, out_hbm.at[pb_v], sem2).wait()
        for _ in range(NB - 1):
            drain()

    return emb


def kernel(labels, embedding_table):
    B = labels.shape[0]
    V, D = embedding_table.shape
    emb = _make_sc_gather(V, D, B)
    labels = labels.astype(jnp.int32)
    perm = jnp.argsort(labels).astype(jnp.int32)
    slab = jnp.take(labels, perm)
    out2 = emb(slab, perm, embedding_table.T)
    return out2[:, :D]


# speedup vs baseline: 1.0026x; 1.0026x over previous
import functools

import jax
import jax.numpy as jnp
from jax import lax
from jax.experimental import pallas as pl
from jax.experimental.pallas import tpu as pltpu, tpu_sc as plsc


def _make_sc_gather(V, D, B):
    info = plsc.get_sparse_core_info()
    L = info.num_lanes
    NW = info.num_cores * info.num_subcores
    assert B % (8 * NW) == 0 and D % L == 0
    b_per_w = B // NW
    NB = 6
    HC = b_per_w // 2
    mesh = plsc.VectorSubcoreMesh(core_axis_name="c", subcore_axis_name="s")

    @functools.partial(
        pl.kernel,
        mesh=mesh,
        compiler_params=pltpu.CompilerParams(needs_layout_passes=False),
        out_type=jax.ShapeDtypeStruct((B, 2 * D), jnp.float32),
        scratch_types=[
            pltpu.VMEM((b_per_w + L,), jnp.int32),
            pltpu.VMEM((b_per_w + L,), jnp.int32),
            pltpu.VMEM((b_per_w + L,), jnp.int32),
            pltpu.VMEM((b_per_w + L,), jnp.int32),
            pltpu.VMEM((HC,), jnp.int32),
            pltpu.VMEM((HC,), jnp.int32),
            pltpu.VMEM((NB, D, 128), jnp.float32),
            pltpu.VMEM((HC, 2 * D), jnp.float32),
            pltpu.SemaphoreType.DMA,
            pltpu.SemaphoreType.DMA,
        ],
    )
    def emb(slab_hbm, perm_hbm, tt_hbm, out_hbm,
            lab_v, labs_v, ord_v, wbase_v, pa_v, pb_v, buf_v, rows_v,
            sem, sem2):
        wid = lax.axis_index("s") * info.num_cores + lax.axis_index("c")
        base = wid * b_per_w
        pltpu.sync_copy(slab_hbm.at[pl.ds(base, b_per_w)],
                        lab_v.at[pl.ds(0, b_per_w)])
        pltpu.sync_copy(slab_hbm.at[pl.ds(base, b_per_w)],
                        labs_v.at[pl.ds(8, b_per_w)])
        pltpu.sync_copy(perm_hbm.at[pl.ds(base, HC)], pa_v)
        pltpu.sync_copy(perm_hbm.at[pl.ds(base + HC, HC)], pb_v)

        def scan_body(g, nwin):
            vec = lab_v[pl.ds(g * L, L)]
            win = lax.shift_right_logical(vec, 7)
            pvec = labs_v[pl.ds(g * L + 7, L)]
            pwin = lax.shift_right_logical(pvec, 7)
            first = jnp.logical_and(g == 0, lax.iota(jnp.int32, L) == 0)
            chg = jnp.logical_or(win != pwin, first)
            inc = jnp.where(chg, jnp.int32(1), jnp.int32(0))
            ord_v[pl.ds(g * L, L)] = nwin - 1 + plsc.cumsum(inc)
            plsc.store_compressed(
                wbase_v.at[pl.ds(nwin, L)], win * 128, mask=chg
            )
            cnt = plsc.all_reduce_population_count(chg)
            return nwin + cnt[0]

        n_win = lax.fori_loop(0, b_per_w // L, scan_body, jnp.int32(0))

        def fire(k):
            kc = jnp.minimum(k, n_win - 1)
            bse = pl.multiple_of(wbase_v[pl.ds(kc, L)][0], 128)
            pltpu.async_copy(
                tt_hbm.at[:, pl.ds(bse, 128)],
                buf_v.at[lax.rem(kc, NB)], sem
            )

        def drain():
            pltpu.make_async_copy(
                tt_hbm.at[:, pl.ds(0, 128)], buf_v.at[0], sem
            ).wait()

        for k in range(NB - 1):
            fire(jnp.int32(k))

        def label_body(i, cur):
            o = ord_v[pl.ds(i, L)][0]

            @pl.when(o != cur)
            def _():
                drain()
                fire(o + NB - 1)

            slot = lax.rem(o, NB)
            col = lab_v[pl.ds(i, L)][0]
            lo16 = jnp.full((L,), lax.rem(col, 128), jnp.int32)
            r = lax.rem(i, HC)
            for q in range(D // L):
                c16 = lax.iota(jnp.int32, L) + q * L
                rows_v[r, pl.ds(q * L, L)] = plsc.load_gather(
                    buf_v.at[slot], [c16, lo16]
                )
            return o

        cur = lax.fori_loop(0, HC, label_body, jnp.int32(-1))
        pltpu.async_copy(rows_v, out_hbm.at[pa_v], sem2).wait()
        cur = lax.fori_loop(HC, b_per_w, label_body, cur)
        pltpu.async_copy(rows_v, out_hbm.at[pb_v], sem2).wait()
        for _ in range(NB - 1):
            drain()

    return emb


def kernel(labels, embedding_table):
    B = labels.shape[0]
    V, D = embedding_table.shape
    emb = _make_sc_gather(V, D, B)
    labels = labels.astype(jnp.int32)
    perm = jnp.argsort(labels).astype(jnp.int32)
    slab = jnp.take(labels, perm)
    out2 = emb(slab, perm, embedding_table.T)
    return out2[:, :D]
